# Initial kernel scaffold; baseline (speedup 1.0000x reference)
import jax
import jax.numpy as jnp
from jax.experimental import pallas as pl


def kernel(x_lex, edge_index, batch, emb, proj_w, proj_b, ggnn_w, gru_wih, gru_whh, gru_bih, gru_bhh, cls_w1, cls_b1, cls_w2, cls_b2):
    raise NotImplementedError("write your pallas kernel here")


# speedup vs baseline: 2.6040x; 2.6040x over previous
import functools

import jax
import jax.numpy as jnp
from jax import lax
from jax.experimental import pallas as pl
from jax.experimental.pallas import tpu as pltpu
from jax.experimental.pallas import tpu_sc as plsc

NC, NS = 2, 16
NW = NC * NS
G = 64


@functools.partial(jax.jit, static_argnums=(2, 3))
def _sc_emb_gather(emb, idx3, K, C):
    D = emb.shape[1]
    BW = K * C
    NP = NW * BW
    mesh = plsc.VectorSubcoreMesh(core_axis_name="c", subcore_axis_name="s")

    @functools.partial(
        pl.kernel,
        mesh=mesh,
        out_type=jax.ShapeDtypeStruct((NP, D), jnp.float32),
        scratch_types=[
            pltpu.VMEM((K, C), jnp.int32),
            pltpu.VMEM((C, D), jnp.float32),
            pltpu.SemaphoreType.DMA,
        ],
    )
    def k(emb_hbm, idx_hbm, out_hbm, idx_v, rows_v, sem):
        cid = lax.axis_index("c")
        sid = lax.axis_index("s")
        wid = sid * NC + cid
        pltpu.sync_copy(idx_hbm.at[wid], idx_v)

        def chunk(j, _):
            pltpu.async_copy(emb_hbm.at[idx_v.at[j]], rows_v, sem).wait()
            pltpu.sync_copy(rows_v, out_hbm.at[pl.ds(wid * BW + j * C, C)])
            return 0

        lax.fori_loop(0, K, chunk, 0)

    return k(emb, idx3)


@functools.partial(jax.jit, static_argnums=(3, 4))
def _sc_edge_agg(m, src, dst, K, C):
    N, H = m.shape
    E = src.shape[0]
    EW = E // NS
    HALF = N // 2
    HP = HALF + 8
    ZB = 312
    ZW = 320
    ZR = 80
    mesh = plsc.VectorSubcoreMesh(core_axis_name="c", subcore_axis_name="s")

    @functools.partial(
        pl.kernel,
        mesh=mesh,
        out_type=jax.ShapeDtypeStruct((NC, HP, H), jnp.float32),
        scratch_types=[
            pltpu.VMEM((C,), jnp.int32),
            pltpu.VMEM((C,), jnp.int32),
            pltpu.VMEM((C,), jnp.int32),
            pltpu.VMEM((C, H), jnp.float32),
            pltpu.VMEM((ZR, H), jnp.float32),
            pltpu.VMEM_SHARED((HP, H), jnp.float32),
            pltpu.SemaphoreType.DMA,
        ],
    )
    def k(m_hbm, src_hbm, dst_hbm, out_hbm, si_v, di_v, di2_v, rows_v, z_v,
          agg_sh, sem):
        cid = lax.axis_index("c")
        sid = lax.axis_index("s")
        base = sid * ZB
        lo = cid * HALF

        nvec = H // 16

        def zrow(i, _):
            z_v[i // nvec, pl.ds((i % nvec) * 16, 16)] = jnp.zeros((16,), jnp.float32)
            return 0

        lax.fori_loop(0, ZR * nvec, zrow, 0)

        def zcp(t, _):
            pltpu.sync_copy(z_v, agg_sh.at[pl.ds(base + t * ZR, ZR)])
            return 0

        lax.fori_loop(0, ZW // ZR, zcp, 0)
        plsc.subcore_barrier()

        def chunk(j, _):
            eb = sid * EW + j * C
            pltpu.sync_copy(src_hbm.at[pl.ds(eb, C)], si_v)
            pltpu.sync_copy(dst_hbm.at[pl.ds(eb, C)], di_v)

            def vfix(v, _):
                d = di_v[pl.ds(v * 16, 16)]
                dl = d - lo
                ok = (dl >= 0) & (dl < HALF)
                di2_v[pl.ds(v * 16, 16)] = jnp.where(ok, dl, HALF)
                return 0

            lax.fori_loop(0, C // 16, vfix, 0)
            pltpu.async_copy(m_hbm.at[si_v], rows_v, sem).wait()
            pltpu.sync_copy(rows_v, agg_sh.at[di2_v], add=True)
            return 0

        lax.fori_loop(0, K, chunk, 0)
        plsc.subcore_barrier()

        pltpu.sync_copy(
            agg_sh.at[pl.ds(base, ZW)], out_hbm.at[cid, pl.ds(base, ZW)]
        )

    return k(m, src, dst)


def _dotT(a, b):
    return lax.dot_general(a, b, (((1,), (1,)), ((), ())),
                           preferred_element_type=jnp.float32)


def _tc_map1(f, x, B=1000):
    N, H = x.shape

    def body(x_ref, o_ref):
        o_ref[...] = f(x_ref[...])

    return pl.pallas_call(
        body,
        grid=(N // B,),
        in_specs=[pl.BlockSpec((B, H), lambda i: (i, 0))],
        out_specs=pl.BlockSpec((B, H), lambda i: (i, 0)),
        out_shape=jax.ShapeDtypeStruct((N, H), jnp.float32),
    )(x)

def _tc_proj(x, proj_w, proj_b2, w0, B=1000):
    N, D = x.shape
    H = proj_w.shape[0]

    def body(x_ref, pw_ref, pb_ref, w0_ref, h_ref, m_ref):
        h = _dotT(x_ref[...], pw_ref[...]) + pb_ref[...]
        h_ref[...] = h
        m_ref[...] = jnp.dot(h, w0_ref[...], preferred_element_type=jnp.float32)

    return pl.pallas_call(
        body,
        grid=(N // B,),
        in_specs=[
            pl.BlockSpec((B, D), lambda i: (i, 0)),
            pl.BlockSpec((H, D), lambda i: (0, 0)),
            pl.BlockSpec((1, H), lambda i: (0, 0)),
            pl.BlockSpec((H, H), lambda i: (0, 0)),
        ],
        out_specs=[
            pl.BlockSpec((B, H), lambda i: (i, 0)),
            pl.BlockSpec((B, H), lambda i: (i, 0)),
        ],
        out_shape=[jax.ShapeDtypeStruct((N, H), jnp.float32)] * 2,
    )(x, proj_w, proj_b2, w0)


def _sigmoid2(x):
    t = jnp.exp(-jnp.abs(x))
    return jnp.where(x >= 0, 1.0 / (1.0 + t), t / (1.0 + t))


def _sigmoid(x):
    return 0.5 * (jnp.tanh(0.5 * x) + 1.0)


def _gru_block(h, agg, wih_ref, whh_ref, bih_ref, bhh_ref):
    H = h.shape[1]
    gi = _dotT(agg, wih_ref[...]) + bih_ref[...]
    gh = _dotT(h, whh_ref[...]) + bhh_ref[...]
    r = jax.nn.sigmoid(gi[:, :H] + gh[:, :H])
    z = jax.nn.sigmoid(gi[:, H:2 * H] + gh[:, H:2 * H])
    n = jnp.tanh(gi[:, 2 * H:] + r * gh[:, 2 * H:])
    return (1.0 - z) * n + z * h


def _tc_gru(h, agg, wih, whh, bih2, bhh2, wnext, B=1000):
    N, H = h.shape

    def body(h_ref, a_ref, wih_ref, whh_ref, bih_ref, bhh_ref, wn_ref,
             ho_ref, mo_ref):
        h2 = _gru_block(h_ref[...], a_ref[...], wih_ref, whh_ref, bih_ref,
                        bhh_ref)
        ho_ref[...] = h2
        mo_ref[...] = jnp.dot(h2, wn_ref[...], preferred_element_type=jnp.float32)

    return pl.pallas_call(
        body,
        grid=(N // B,),
        in_specs=[
            pl.BlockSpec((B, H), lambda i: (i, 0)),
            pl.BlockSpec((B, H), lambda i: (i, 0)),
            pl.BlockSpec((3 * H, H), lambda i: (0, 0)),
            pl.BlockSpec((3 * H, H), lambda i: (0, 0)),
            pl.BlockSpec((1, 3 * H), lambda i: (0, 0)),
            pl.BlockSpec((1, 3 * H), lambda i: (0, 0)),
            pl.BlockSpec((H, H), lambda i: (0, 0)),
        ],
        out_specs=[
            pl.BlockSpec((B, H), lambda i: (i, 0)),
            pl.BlockSpec((B, H), lambda i: (i, 0)),
        ],
        out_shape=[jax.ShapeDtypeStruct((N, H), jnp.float32)] * 2,
    )(h, agg, wih, whh, bih2, bhh2, wnext)


def _tc_gru_pool_cls(h, agg, wih, whh, bih2, bhh2, batch2,
                     cls_w1, cls_b12, cls_w2, cls_b22, B=1000):
    N, H = h.shape
    HH = cls_w1.shape[0]
    nblk = N // B

    def body(h_ref, a_ref, wih_ref, whh_ref, bih_ref, bhh_ref, bat_ref,
             w1_ref, b1_ref, w2_ref, b2_ref, logit_ref, pool_ref, acc):
        i = pl.program_id(0)

        @pl.when(i == 0)
        def _():
            acc[...] = jnp.full((G, H), -jnp.inf, jnp.float32)

        h2 = _gru_block(h_ref[...], a_ref[...], wih_ref, whh_ref, bih_ref,
                        bhh_ref)
        bat = bat_ref[...]
        for g in range(G):
            mx = jnp.max(jnp.where(bat == g, h2, -jnp.inf), axis=0,
                         keepdims=True)
            acc[pl.ds(g, 1), :] = jnp.maximum(acc[pl.ds(g, 1), :], mx)

        @pl.when(i == nblk - 1)
        def _():
            pool = acc[...]
            pool_ref[...] = pool
            hid = jax.nn.relu(_dotT(pool, w1_ref[...]) + b1_ref[...])
            logit_ref[...] = (jnp.sum(hid * w2_ref[...], axis=1, keepdims=True)
                              + b2_ref[0, 0])

    return pl.pallas_call(
        body,
        grid=(nblk,),
        in_specs=[
            pl.BlockSpec((B, H), lambda i: (i, 0)),
            pl.BlockSpec((B, H), lambda i: (i, 0)),
            pl.BlockSpec((3 * H, H), lambda i: (0, 0)),
            pl.BlockSpec((3 * H, H), lambda i: (0, 0)),
            pl.BlockSpec((1, 3 * H), lambda i: (0, 0)),
            pl.BlockSpec((1, 3 * H), lambda i: (0, 0)),
            pl.BlockSpec((B, H), lambda i: (i, 0)),
            pl.BlockSpec((HH, H), lambda i: (0, 0)),
            pl.BlockSpec((1, HH), lambda i: (0, 0)),
            pl.BlockSpec((1, HH), lambda i: (0, 0)),
            pl.BlockSpec((1, 1), lambda i: (0, 0)),
        ],
        out_specs=[
            pl.BlockSpec((G, 1), lambda i: (0, 0)),
            pl.BlockSpec((G, H), lambda i: (0, 0)),
        ],
        out_shape=[
            jax.ShapeDtypeStruct((G, 1), jnp.float32),
            jax.ShapeDtypeStruct((G, H), jnp.float32),
        ],
        scratch_shapes=[pltpu.VMEM((G, H), jnp.float32)],
    )(h, agg, wih, whh, bih2, bhh2, batch2, cls_w1, cls_b12, cls_w2, cls_b22)


def kernel(x_lex, edge_index, batch, emb, proj_w, proj_b, ggnn_w,
           gru_wih, gru_whh, gru_bih, gru_bhh, cls_w1, cls_b1, cls_w2, cls_b2):
    N = x_lex.shape[0]
    E = edge_index.shape[1]
    H = proj_w.shape[0]
    L = ggnn_w.shape[0]

    x_lex = x_lex.astype(jnp.int32)
    edge_index = edge_index.astype(jnp.int32)
    batch = batch.astype(jnp.int32)

    CG = 64
    BW = -(-N // (NW * CG)) * CG
    NP = NW * BW
    idx_pad = jnp.pad(x_lex, (0, NP - N))
    idx3 = idx_pad.reshape(NW, BW // CG, CG)
    x = _sc_emb_gather(emb, idx3, BW // CG, CG)[:N]

    CE = 80
    KE = E // (NS * CE)
    src1 = edge_index[0]
    dst1 = edge_index[1]

    proj_b2 = proj_b.reshape(1, H)
    bih2 = gru_bih.reshape(1, 3 * H)
    bhh2 = gru_bhh.reshape(1, 3 * H)
    batch2 = jnp.broadcast_to(batch.reshape(N, 1), (N, H))
    cls_b12 = cls_b1.reshape(1, -1)
    cls_b22 = cls_b2.reshape(1, 1)

    h, m = _tc_proj(x, proj_w, proj_b2, ggnn_w[0])
    for i in range(L):
        aggp = _sc_edge_agg(m, src1, dst1, KE, CE)
        agg = jnp.concatenate([aggp[0, :N // 2], aggp[1, :N // 2]], axis=0)
        if i < L - 1:
            h, m = _tc_gru(h, agg, gru_wih, gru_whh, bih2, bhh2, ggnn_w[i + 1])
        else:
            logits, pool = _tc_gru_pool_cls(
                h, agg, gru_wih, gru_whh, bih2, bhh2, batch2,
                cls_w1, cls_b12, cls_w2, cls_b22)
    return (logits, pool)
